# baseline (speedup 1.0000x reference)
import functools

import jax
import jax.numpy as jnp
from jax import lax
from jax.experimental import pallas as pl
from jax.experimental.pallas import tpu as pltpu
from jax.experimental.pallas import tpu_sc as plsc

NC, NS, L = 2, 16, 16
NW = NC * NS
B = 16384
EMB = 16
BPW = B // NW
CH = 128
NFCH = BPW // CH


def _sc_body(fidx_hbm, f0_hbm, f1_hbm, f2_hbm, *rest):
    emb_hbm = rest[:EMB]
    out_hbm = rest[EMB]
    fidx_v, vidx_v, rows_v, sem, sem2 = rest[EMB + 1:]
    faces_hbm = (f0_hbm, f1_hbm, f2_hbm)

    wid = lax.axis_index("s") * NC + lax.axis_index("c")
    base = wid * BPW

    pltpu.sync_copy(fidx_hbm.at[pl.ds(wid * NFCH, NFCH)], fidx_v)

    face_copies = [
        pltpu.make_async_copy(
            faces_hbm[v].at[fidx_v.at[j]],
            vidx_v.at[v, j],
            sem,
        )
        for v in range(3)
        for j in range(NFCH)
    ]
    for c in face_copies:
        c.start()
    for c in face_copies:
        c.wait()

    emb_copies = [
        pltpu.make_async_copy(
            emb_hbm[e].at[vidx_v.at[v, j]],
            rows_v.at[e, pl.ds(v * BPW + j * CH, CH)],
            sem2,
        )
        for e in range(EMB)
        for v in range(3)
        for j in range(NFCH)
    ]
    for c in emb_copies:
        c.start()
    for c in emb_copies:
        c.wait()

    for v in range(3):
        for e in range(EMB):
            pltpu.sync_copy(rows_v.at[e, pl.ds(v * BPW, BPW)],
                            out_hbm.at[v, e, pl.ds(base, BPW)])


@jax.jit
def _sc_gather(fidx2d, f0, f1, f2, *emb_cols):
    mesh = plsc.VectorSubcoreMesh(
        core_axis_name="c", subcore_axis_name="s",
        num_cores=NC, num_subcores=NS)
    return pl.kernel(
        _sc_body,
        out_type=jax.ShapeDtypeStruct((3, EMB, B), jnp.float32),
        mesh=mesh,
        scratch_types=[
            pltpu.VMEM((NFCH, CH), jnp.int32),
            pltpu.VMEM((3, NFCH, CH), jnp.int32),
            pltpu.VMEM((EMB, 3 * BPW), jnp.float32),
            pltpu.SemaphoreType.DMA,
            pltpu.SemaphoreType.DMA,
        ],
        compiler_params=pltpu.CompilerParams(
            needs_layout_passes=False, use_tc_tiling_on_sc=False),
    )(fidx2d, f0, f1, f2, *emb_cols)


NBUF = 4
NCHUNK = 8


def _chunk_plan(total):
    step = (total // NCHUNK) // 128 * 128
    offs = [i * step for i in range(NCHUNK)]
    sizes = [step] * (NCHUNK - 1) + [total - (NCHUNK - 1) * step]
    return list(zip(offs, sizes))


def _extract_body(rb, nrb, src_hbm, *rest):
    outs = rest[:rb * nrb]
    bufs = rest[rb * nrb:rb * nrb + NBUF]
    isems = rest[rb * nrb + NBUF:rb * nrb + 2 * NBUF]
    osem = rest[rb * nrb + 2 * NBUF]
    units = [(r, off, n)
             for r in range(nrb)
             for off, n in _chunk_plan(src_hbm.shape[1])]

    def in_copy(i):
        r, off, n = units[i]
        return pltpu.make_async_copy(
            src_hbm.at[pl.ds(r * rb, rb), pl.ds(off, n)],
            bufs[i % NBUF].at[pl.ds(0, rb), pl.ds(0, n)],
            isems[i % NBUF])

    def out_copies(i):
        r, off, n = units[i]
        return [pltpu.make_async_copy(
                    bufs[i % NBUF].at[k, pl.ds(0, n)],
                    outs[r * rb + k].at[pl.ds(off, n)],
                    osem)
                for k in range(rb)]

    nu = len(units)
    pending = []
    for i in range(min(NBUF, nu)):
        in_copy(i).start()
    for i in range(nu):
        in_copy(i).wait()
        oc = out_copies(i)
        for c in oc:
            c.start()
        pending.append(oc)
        if i + NBUF < nu:
            for c in pending[i]:
                c.wait()
            in_copy(i + NBUF).start()
        elif i >= nu - NBUF:
            for c in pending[i]:
                c.wait()


def _extract_cols(srcT, rb):
    nf, n = srcT.shape
    nrb = nf // rb
    maxn = _chunk_plan(n)[-1][1]
    return pl.pallas_call(
        functools.partial(_extract_body, rb, nrb),
        out_shape=[jax.ShapeDtypeStruct((n,), srcT.dtype)] * nf,
        in_specs=[pl.BlockSpec(memory_space=pl.ANY)],
        out_specs=[pl.BlockSpec(memory_space=pl.ANY)] * nf,
        scratch_shapes=(
            [pltpu.VMEM((8, maxn), srcT.dtype)] * NBUF
            + [pltpu.SemaphoreType.DMA] * NBUF
            + [pltpu.SemaphoreType.DMA]
        ),
    )(srcT)


@jax.jit
def _extract_emb(embT):
    return _extract_cols(embT, 8)


@jax.jit
def _extract_faces(facesT):
    return _extract_cols(facesT, 3)


def _mlp_body(rows_ref, bary_ref, w1_ref, b1_ref, w2_ref, b2_ref,
              w3_ref, b3_ref, o_ref):
    r = rows_ref[...]
    w = bary_ref[...]
    xt = (w[0:1, :] * r[0] + w[1:2, :] * r[1] + w[2:3, :] * r[2])
    dn = (((0,), (0,)), ((), ()))
    h = jnp.maximum(
        lax.dot_general(w1_ref[...], xt, dn,
                        preferred_element_type=jnp.float32)
        + b1_ref[...], 0.0)
    h = jnp.maximum(
        lax.dot_general(w2_ref[...], h, dn,
                        preferred_element_type=jnp.float32)
        + b2_ref[...], 0.0)
    z = (lax.dot_general(w3_ref[...], h, dn,
                         preferred_element_type=jnp.float32)
         + b3_ref[...])
    o_ref[...] = jax.nn.sigmoid(z)


@jax.jit
def _tc_blend_mlp(rows, baryT, W1, b1, W2, b2, W3, b3):
    nblk = 4
    blk = B // nblk
    return pl.pallas_call(
        _mlp_body,
        out_shape=jax.ShapeDtypeStruct((3, B), jnp.float32),
        grid=(nblk,),
        in_specs=[
            pl.BlockSpec((3, EMB, blk), lambda i: (0, 0, i)),
            pl.BlockSpec((3, blk), lambda i: (0, i)),
            pl.BlockSpec((EMB, 32), lambda i: (0, 0)),
            pl.BlockSpec((32, 1), lambda i: (0, 0)),
            pl.BlockSpec((32, 32), lambda i: (0, 0)),
            pl.BlockSpec((32, 1), lambda i: (0, 0)),
            pl.BlockSpec((32, 3), lambda i: (0, 0)),
            pl.BlockSpec((3, 1), lambda i: (0, 0)),
        ],
        out_specs=pl.BlockSpec((3, blk), lambda i: (0, i)),
    )(rows, baryT, W1, b1.reshape(32, 1), W2, b2.reshape(32, 1),
      W3, b3.reshape(3, 1))


def kernel(barycentrics, face_idx, faces, embeddings, W1, b1, W2, b2, W3, b3):
    fidx2d = face_idx.astype(jnp.int32).reshape(NW * NFCH, CH)
    f_cols = _extract_faces(faces.T)
    e_cols = _extract_emb(embeddings.T)
    rows = _sc_gather(fidx2d, *f_cols, *e_cols)
    out_t = _tc_blend_mlp(rows, barycentrics.T, W1, b1, W2, b2, W3, b3)
    return out_t.T


# speedup vs baseline: 1.0063x; 1.0063x over previous
import functools

import jax
import jax.numpy as jnp
from jax import lax
from jax.experimental import pallas as pl
from jax.experimental.pallas import tpu as pltpu
from jax.experimental.pallas import tpu_sc as plsc

NC, NS, L = 2, 16, 16
NW = NC * NS
B = 16384
EMB = 16
BPW = B // NW
CH = 128
NFCH = BPW // CH


def _sc1_body(fidx_hbm, f0_hbm, f1_hbm, f2_hbm, out_hbm,
              fidx_v, vidx_v, sem):
    faces_hbm = (f0_hbm, f1_hbm, f2_hbm)
    wid = lax.axis_index("s") * NC + lax.axis_index("c")
    base = wid * BPW

    pltpu.sync_copy(fidx_hbm.at[pl.ds(wid * NFCH, NFCH)], fidx_v)

    face_copies = [
        pltpu.make_async_copy(
            faces_hbm[v].at[fidx_v.at[j]],
            vidx_v.at[v, pl.ds(j * CH, CH)],
            sem,
        )
        for v in range(3)
        for j in range(NFCH)
    ]
    for c in face_copies:
        c.start()
    for c in face_copies:
        c.wait()

    for v in range(3):
        pltpu.sync_copy(vidx_v.at[v], out_hbm.at[v, pl.ds(base, BPW)])


@jax.jit
def _sc_face_gather(fidx2d, f0, f1, f2):
    mesh = plsc.VectorSubcoreMesh(
        core_axis_name="c", subcore_axis_name="s",
        num_cores=NC, num_subcores=NS)
    return pl.kernel(
        _sc1_body,
        out_type=jax.ShapeDtypeStruct((3, B), jnp.int32),
        mesh=mesh,
        scratch_types=[
            pltpu.VMEM((NFCH, CH), jnp.int32),
            pltpu.VMEM((3, BPW), jnp.int32),
            pltpu.SemaphoreType.DMA,
        ],
        compiler_params=pltpu.CompilerParams(
            needs_layout_passes=False, use_tc_tiling_on_sc=False),
    )(fidx2d, f0, f1, f2)


def _sc2_body(vidx_hbm, *rest):
    emb_hbm = rest[:EMB]
    out_hbm = rest[EMB]
    vidx_v, rows_v, sem2 = rest[EMB + 1:]

    wid = lax.axis_index("s") * NC + lax.axis_index("c")
    base = wid * BPW

    pltpu.sync_copy(vidx_hbm.at[:, pl.ds(base, BPW)], vidx_v)

    emb_copies = [
        pltpu.make_async_copy(
            emb_hbm[e].at[vidx_v.at[v, pl.ds(j * CH, CH)]],
            rows_v.at[e, pl.ds(v * BPW + j * CH, CH)],
            sem2,
        )
        for e in range(EMB)
        for v in range(3)
        for j in range(NFCH)
    ]
    for c in emb_copies:
        c.start()
    for c in emb_copies:
        c.wait()

    for v in range(3):
        for e in range(EMB):
            pltpu.sync_copy(rows_v.at[e, pl.ds(v * BPW, BPW)],
                            out_hbm.at[v, e, pl.ds(base, BPW)])


@jax.jit
def _sc_emb_gather(vidx, *emb_cols):
    mesh = plsc.VectorSubcoreMesh(
        core_axis_name="c", subcore_axis_name="s",
        num_cores=NC, num_subcores=NS)
    return pl.kernel(
        _sc2_body,
        out_type=jax.ShapeDtypeStruct((3, EMB, B), jnp.float32),
        mesh=mesh,
        scratch_types=[
            pltpu.VMEM((3, BPW), jnp.int32),
            pltpu.VMEM((EMB, 3 * BPW), jnp.float32),
            pltpu.SemaphoreType.DMA,
        ],
        compiler_params=pltpu.CompilerParams(
            needs_layout_passes=False, use_tc_tiling_on_sc=False),
    )(vidx, *emb_cols)


NBUF = 4
NCHUNK = 8


def _chunk_plan(total):
    step = (total // NCHUNK) // 128 * 128
    offs = [i * step for i in range(NCHUNK)]
    sizes = [step] * (NCHUNK - 1) + [total - (NCHUNK - 1) * step]
    return list(zip(offs, sizes))


def _extract_body(rb, nrb, src_hbm, *rest):
    outs = rest[:rb * nrb]
    bufs = rest[rb * nrb:rb * nrb + NBUF]
    isems = rest[rb * nrb + NBUF:rb * nrb + 2 * NBUF]
    osem = rest[rb * nrb + 2 * NBUF]
    units = [(r, off, n)
             for r in range(nrb)
             for off, n in _chunk_plan(src_hbm.shape[1])]

    def in_copy(i):
        r, off, n = units[i]
        return pltpu.make_async_copy(
            src_hbm.at[pl.ds(r * rb, rb), pl.ds(off, n)],
            bufs[i % NBUF].at[pl.ds(0, rb), pl.ds(0, n)],
            isems[i % NBUF])

    def out_copies(i):
        r, off, n = units[i]
        return [pltpu.make_async_copy(
                    bufs[i % NBUF].at[k, pl.ds(0, n)],
                    outs[r * rb + k].at[pl.ds(off, n)],
                    osem)
                for k in range(rb)]

    nu = len(units)
    pending = []
    for i in range(min(NBUF, nu)):
        in_copy(i).start()
    for i in range(nu):
        in_copy(i).wait()
        oc = out_copies(i)
        for c in oc:
            c.start()
        pending.append(oc)
        if i + NBUF < nu:
            for c in pending[i]:
                c.wait()
            in_copy(i + NBUF).start()
        elif i >= nu - NBUF:
            for c in pending[i]:
                c.wait()


def _extract_cols(srcT, rb):
    nf, n = srcT.shape
    nrb = nf // rb
    maxn = _chunk_plan(n)[-1][1]
    return pl.pallas_call(
        functools.partial(_extract_body, rb, nrb),
        out_shape=[jax.ShapeDtypeStruct((n,), srcT.dtype)] * nf,
        in_specs=[pl.BlockSpec(memory_space=pl.ANY)],
        out_specs=[pl.BlockSpec(memory_space=pl.ANY)] * nf,
        scratch_shapes=(
            [pltpu.VMEM((8, maxn), srcT.dtype)] * NBUF
            + [pltpu.SemaphoreType.DMA] * NBUF
            + [pltpu.SemaphoreType.DMA]
        ),
    )(srcT)


@jax.jit
def _extract_emb(embT):
    return _extract_cols(embT, 8)


@jax.jit
def _extract_faces(facesT):
    return _extract_cols(facesT, 3)


def _mlp_body(rows_ref, bary_ref, w1_ref, b1_ref, w2_ref, b2_ref,
              w3_ref, b3_ref, o_ref):
    r = rows_ref[...]
    w = bary_ref[...]
    xt = (w[0:1, :] * r[0] + w[1:2, :] * r[1] + w[2:3, :] * r[2])
    dn = (((0,), (0,)), ((), ()))
    h = jnp.maximum(
        lax.dot_general(w1_ref[...], xt, dn,
                        preferred_element_type=jnp.float32)
        + b1_ref[...], 0.0)
    h = jnp.maximum(
        lax.dot_general(w2_ref[...], h, dn,
                        preferred_element_type=jnp.float32)
        + b2_ref[...], 0.0)
    z = (lax.dot_general(w3_ref[...], h, dn,
                         preferred_element_type=jnp.float32)
         + b3_ref[...])
    o_ref[...] = jax.nn.sigmoid(z)


@jax.jit
def _tc_blend_mlp(rows, baryT, W1, b1, W2, b2, W3, b3):
    nblk = 4
    blk = B // nblk
    return pl.pallas_call(
        _mlp_body,
        out_shape=jax.ShapeDtypeStruct((3, B), jnp.float32),
        grid=(nblk,),
        in_specs=[
            pl.BlockSpec((3, EMB, blk), lambda i: (0, 0, i)),
            pl.BlockSpec((3, blk), lambda i: (0, i)),
            pl.BlockSpec((EMB, 32), lambda i: (0, 0)),
            pl.BlockSpec((32, 1), lambda i: (0, 0)),
            pl.BlockSpec((32, 32), lambda i: (0, 0)),
            pl.BlockSpec((32, 1), lambda i: (0, 0)),
            pl.BlockSpec((32, 3), lambda i: (0, 0)),
            pl.BlockSpec((3, 1), lambda i: (0, 0)),
        ],
        out_specs=pl.BlockSpec((3, blk), lambda i: (0, i)),
    )(rows, baryT, W1, b1.reshape(32, 1), W2, b2.reshape(32, 1),
      W3, b3.reshape(3, 1))


def kernel(barycentrics, face_idx, faces, embeddings, W1, b1, W2, b2, W3, b3):
    fidx2d = face_idx.astype(jnp.int32).reshape(NW * NFCH, CH)
    f_cols = _extract_faces(faces.T)
    vidx = _sc_face_gather(fidx2d, *f_cols)
    e_cols = _extract_emb(embeddings.T)
    rows = _sc_emb_gather(vidx, *e_cols)
    out_t = _tc_blend_mlp(rows, barycentrics.T, W1, b1, W2, b2, W3, b3)
    return out_t.T
